# baseline (speedup 1.0000x reference)
import functools

import jax
import jax.numpy as jnp
from jax.experimental import pallas as pl

_TAU = 0.05


def _feats_body(om2_ref, h_ref, b2c_ref, a2r_ref, sh_ref,
                f_ref, *, nh, no, kqs, ns):
    for s in range(ns):
        om2 = om2_ref[s]
        oT = jnp.transpose(om2) * -0.5
        hT = jnp.transpose(h_ref[s])
        f_ref[s] = _one_sample(om2, oT, hT, b2c_ref[s],
                               a2r_ref[s], sh_ref[s], nh, no, kqs)


def _mlp_body(f_ref, w1_ref, b1_ref, w2_ref, b2_ref, out_ref):
    hid = jnp.maximum(
        jnp.dot(f_ref[...], w1_ref[...], preferred_element_type=jnp.float32)
        + b1_ref[...], 0.0)
    out_ref[...] = (
        jnp.dot(hid, w2_ref[...], preferred_element_type=jnp.float32)
        + b2_ref[...])


def _one_sample(om2, oT, hT, b2c, a2r, shr, nh, no, kqs):
    gT2 = jnp.dot(om2, hT, preferred_element_type=jnp.float32)
    sqT = (b2c + a2r) + gT2

    sqmin_h = jnp.min(sqT, axis=0, keepdims=True)
    sqmin_o = jnp.min(sqT, axis=1, keepdims=True)
    dmin_h = jnp.sqrt(jnp.maximum(sqmin_h, 1e-12))
    dmin_o = jnp.sqrt(jnp.maximum(jnp.transpose(sqmin_o), 1e-12))

    eqf = (sqT == sqmin_h).astype(jnp.float32)
    o_nn = jnp.dot(oT, eqf, preferred_element_type=jnp.float32)
    cnt = jnp.dot(jnp.ones((1, no), jnp.float32), eqf,
                  preferred_element_type=jnp.float32)

    vecT = o_nn / cnt - hT
    nrm = jnp.sqrt(jnp.maximum(
        jnp.sum(vecT * vecT, axis=0, keepdims=True), 1e-6))
    dir_sum = jnp.sum(vecT / nrm, axis=1, keepdims=True)
    dir_mean = jnp.transpose(dir_sum) * (1.0 / nh)

    w_h = jnp.exp(dmin_h * (-1.0 / _TAU)) * shr

    dm_col = jnp.transpose(dmin_h)
    ones_row = jnp.ones((1, nh), jnp.float32)
    lt = (dm_col < dmin_h).astype(jnp.float32)
    le = (dm_col <= dmin_h).astype(jnp.float32)
    r1 = jnp.dot(ones_row, lt, preferred_element_type=jnp.float32)
    rle = jnp.dot(ones_row, le, preferred_element_type=jnp.float32)
    inv_e = 1.0 / (rle - r1)

    qmeans = []
    for kq in kqs:
        frac = jnp.clip((kq - r1) * inv_e, 0.0, 1.0)
        qmeans.append(
            jnp.sum(dmin_h * frac, axis=1, keepdims=True) * (1.0 / kq))

    mean_dh = jnp.sum(dmin_h, axis=1, keepdims=True) * (1.0 / nh)
    min_dh = jnp.min(dmin_h, axis=1, keepdims=True)
    mean_wh = jnp.sum(w_h, axis=1, keepdims=True) * (1.0 / nh)
    mean_do = jnp.sum(dmin_o, axis=1, keepdims=True) * (1.0 / no)

    return jnp.concatenate(
        [mean_dh, min_dh, qmeans[0], qmeans[1], qmeans[2],
         mean_wh, dir_mean, mean_do], axis=1)


def kernel(human_bt_n3, object_bt_m3, s_h_bt_n, s_o_bt_m, W1, b1, W2, b2):
    B, T, Nh, _ = human_bt_n3.shape
    No = object_bt_m3.shape[2]
    BT = B * T
    h = human_bt_n3.reshape(BT, Nh, 3)
    o = object_bt_m3.reshape(BT, No, 3)
    om2 = -2.0 * o
    a2r = jnp.sum(h * h, axis=2)[:, None, :]
    b2c = jnp.sum(o * o, axis=2)[:, :, None]
    shr = s_h_bt_n.reshape(BT, 1, Nh)
    kqs = tuple(int(max(1, round(q * Nh))) for q in (0.2, 0.5, 0.8))

    NS = 8
    feats = pl.pallas_call(
        functools.partial(_feats_body, nh=Nh, no=No, kqs=kqs, ns=NS),
        grid=(BT // NS,),
        in_specs=[
            pl.BlockSpec((NS, No, 3), lambda i: (i, 0, 0)),
            pl.BlockSpec((NS, Nh, 3), lambda i: (i, 0, 0)),
            pl.BlockSpec((NS, No, 1), lambda i: (i, 0, 0)),
            pl.BlockSpec((NS, 1, Nh), lambda i: (i, 0, 0)),
            pl.BlockSpec((NS, 1, Nh), lambda i: (i, 0, 0)),
        ],
        out_specs=pl.BlockSpec((NS, 1, 10), lambda i: (i, 0, 0)),
        out_shape=jax.ShapeDtypeStruct((BT, 1, 10), jnp.float32),
    )(om2, h, b2c, a2r, shr)
    feats = feats.reshape(BT, 10)

    H = W1.shape[1]
    F = W2.shape[1]
    out = pl.pallas_call(
        _mlp_body,
        in_specs=[pl.BlockSpec(feats.shape, lambda: (0, 0)),
                  pl.BlockSpec(W1.shape, lambda: (0, 0)),
                  pl.BlockSpec((1, H), lambda: (0, 0)),
                  pl.BlockSpec(W2.shape, lambda: (0, 0)),
                  pl.BlockSpec((1, F), lambda: (0, 0))],
        out_specs=pl.BlockSpec((BT, F), lambda: (0, 0)),
        out_shape=jax.ShapeDtypeStruct((BT, F), jnp.float32),
    )(feats, W1, b1.reshape(1, H), W2, b2.reshape(1, F))
    return out.reshape(B, T, F)


# speedup vs baseline: 1.3449x; 1.3449x over previous
import functools

import jax
import jax.numpy as jnp
from jax.experimental import pallas as pl

_TAU = 0.05


def _feats_body(om2_ref, oT_ref, hT_ref, b2c_ref, a2r_ref, sh_ref,
                f_ref, *, nh, no, kqs, ns):
    for s in range(ns):
        f_ref[s] = _one_sample(om2_ref[s], oT_ref[s], hT_ref[s], b2c_ref[s],
                               a2r_ref[s], sh_ref[s], nh, no, kqs)


def _mlp_body(f_ref, w1_ref, b1_ref, w2_ref, b2_ref, out_ref):
    hid = jnp.maximum(
        jnp.dot(f_ref[...], w1_ref[...], preferred_element_type=jnp.float32)
        + b1_ref[...], 0.0)
    out_ref[...] = (
        jnp.dot(hid, w2_ref[...], preferred_element_type=jnp.float32)
        + b2_ref[...])


def _one_sample(om2, oT, hT, b2c, a2r, shr, nh, no, kqs):
    gT2 = jnp.dot(om2, hT, preferred_element_type=jnp.float32)
    sqT = (b2c + a2r) + gT2

    sqmin_h = jnp.min(sqT, axis=0, keepdims=True)
    sqmin_o = jnp.min(sqT, axis=1, keepdims=True)
    dmin_h = jnp.sqrt(jnp.maximum(sqmin_h, 1e-12))
    dmin_o = jnp.sqrt(jnp.maximum(jnp.transpose(sqmin_o), 1e-12))

    eqf = (sqT == sqmin_h).astype(jnp.float32)
    o_nn = jnp.dot(oT, eqf, preferred_element_type=jnp.float32)
    cnt = jnp.dot(jnp.ones((1, no), jnp.float32), eqf,
                  preferred_element_type=jnp.float32)

    vecT = o_nn / cnt - hT
    nrm = jnp.sqrt(jnp.maximum(
        jnp.sum(vecT * vecT, axis=0, keepdims=True), 1e-6))
    dir_sum = jnp.sum(vecT / nrm, axis=1, keepdims=True)
    dir_mean = jnp.transpose(dir_sum) * (1.0 / nh)

    w_h = jnp.exp(dmin_h * (-1.0 / _TAU)) * shr

    dm_col = jnp.transpose(dmin_h)
    ones_row = jnp.ones((1, nh), jnp.float32)
    lt = (dm_col < dmin_h).astype(jnp.float32)
    le = (dm_col <= dmin_h).astype(jnp.float32)
    r1 = jnp.dot(ones_row, lt, preferred_element_type=jnp.float32)
    rle = jnp.dot(ones_row, le, preferred_element_type=jnp.float32)
    inv_e = 1.0 / (rle - r1)

    qmeans = []
    for kq in kqs:
        frac = jnp.clip((kq - r1) * inv_e, 0.0, 1.0)
        qmeans.append(
            jnp.sum(dmin_h * frac, axis=1, keepdims=True) * (1.0 / kq))

    mean_dh = jnp.sum(dmin_h, axis=1, keepdims=True) * (1.0 / nh)
    min_dh = jnp.min(dmin_h, axis=1, keepdims=True)
    mean_wh = jnp.sum(w_h, axis=1, keepdims=True) * (1.0 / nh)
    mean_do = jnp.sum(dmin_o, axis=1, keepdims=True) * (1.0 / no)

    return jnp.concatenate(
        [mean_dh, min_dh, qmeans[0], qmeans[1], qmeans[2],
         mean_wh, dir_mean, mean_do], axis=1)


def kernel(human_bt_n3, object_bt_m3, s_h_bt_n, s_o_bt_m, W1, b1, W2, b2):
    B, T, Nh, _ = human_bt_n3.shape
    No = object_bt_m3.shape[2]
    BT = B * T
    h = human_bt_n3.reshape(BT, Nh, 3)
    o = object_bt_m3.reshape(BT, No, 3)
    hT = h.transpose(0, 2, 1)
    oT = o.transpose(0, 2, 1)
    om2 = -2.0 * o
    a2r = jnp.sum(h * h, axis=2)[:, None, :]
    b2c = jnp.sum(o * o, axis=2)[:, :, None]
    shr = s_h_bt_n.reshape(BT, 1, Nh)
    kqs = tuple(int(max(1, round(q * Nh))) for q in (0.2, 0.5, 0.8))

    NS = 16
    feats = pl.pallas_call(
        functools.partial(_feats_body, nh=Nh, no=No, kqs=kqs, ns=NS),
        grid=(BT // NS,),
        in_specs=[
            pl.BlockSpec((NS, No, 3), lambda i: (i, 0, 0)),
            pl.BlockSpec((NS, 3, No), lambda i: (i, 0, 0)),
            pl.BlockSpec((NS, 3, Nh), lambda i: (i, 0, 0)),
            pl.BlockSpec((NS, No, 1), lambda i: (i, 0, 0)),
            pl.BlockSpec((NS, 1, Nh), lambda i: (i, 0, 0)),
            pl.BlockSpec((NS, 1, Nh), lambda i: (i, 0, 0)),
        ],
        out_specs=pl.BlockSpec((NS, 1, 10), lambda i: (i, 0, 0)),
        out_shape=jax.ShapeDtypeStruct((BT, 1, 10), jnp.float32),
    )(om2, oT, hT, b2c, a2r, shr)
    feats = feats.reshape(BT, 10)

    H = W1.shape[1]
    F = W2.shape[1]
    out = pl.pallas_call(
        _mlp_body,
        in_specs=[pl.BlockSpec(feats.shape, lambda: (0, 0)),
                  pl.BlockSpec(W1.shape, lambda: (0, 0)),
                  pl.BlockSpec((1, H), lambda: (0, 0)),
                  pl.BlockSpec(W2.shape, lambda: (0, 0)),
                  pl.BlockSpec((1, F), lambda: (0, 0))],
        out_specs=pl.BlockSpec((BT, F), lambda: (0, 0)),
        out_shape=jax.ShapeDtypeStruct((BT, F), jnp.float32),
    )(feats, W1, b1.reshape(1, H), W2, b2.reshape(1, F))
    return out.reshape(B, T, F)


# speedup vs baseline: 1.3862x; 1.0307x over previous
import functools

import jax
import jax.numpy as jnp
from jax.experimental import pallas as pl

_TAU = 0.05


def _feats_body(o_ref, hoT_ref, b2c_ref, sh_ref,
                f_ref, *, nh, no, kqs, ns):
    for s in range(ns):
        f_ref[s] = _one_sample(o_ref[s], hoT_ref[s], b2c_ref[s],
                               sh_ref[s], nh, no, kqs)


def _mlp_body(f_ref, w1_ref, b1_ref, w2_ref, b2_ref, out_ref):
    hid = jnp.maximum(
        jnp.dot(f_ref[...], w1_ref[...], preferred_element_type=jnp.float32)
        + b1_ref[...], 0.0)
    out_ref[...] = (
        jnp.dot(hid, w2_ref[...], preferred_element_type=jnp.float32)
        + b2_ref[...])


def _one_sample(o, hoTm2, b2c, shr, nh, no, kqs):
    hTm2 = hoTm2[:, :nh]
    oTm2 = hoTm2[:, nh:]
    a2r = jnp.sum(hTm2 * hTm2, axis=0, keepdims=True) * 0.25
    gT2 = jnp.dot(o, hTm2, preferred_element_type=jnp.float32)
    sqT = (b2c + a2r) + gT2

    sqmin_h = jnp.min(sqT, axis=0, keepdims=True)
    sqmin_o = jnp.min(sqT, axis=1, keepdims=True)
    dmin_h = jnp.sqrt(jnp.maximum(sqmin_h, 1e-12))
    dmin_o = jnp.sqrt(jnp.maximum(jnp.transpose(sqmin_o), 1e-12))

    eqf = (sqT == sqmin_h).astype(jnp.float32)
    o_nn = jnp.dot(oTm2, eqf,
                   preferred_element_type=jnp.float32) * -0.5
    cnt = jnp.dot(jnp.ones((1, no), jnp.float32), eqf,
                  preferred_element_type=jnp.float32)

    vecT = o_nn / cnt - hTm2 * -0.5
    nrm = jnp.sqrt(jnp.maximum(
        jnp.sum(vecT * vecT, axis=0, keepdims=True), 1e-6))
    dir_sum = jnp.sum(vecT / nrm, axis=1, keepdims=True)
    dir_mean = jnp.transpose(dir_sum) * (1.0 / nh)

    w_h = jnp.exp(dmin_h * (-1.0 / _TAU)) * shr

    dm_col = jnp.transpose(dmin_h)
    ones_row = jnp.ones((1, nh), jnp.float32)
    lt = (dm_col < dmin_h).astype(jnp.float32)
    le = (dm_col <= dmin_h).astype(jnp.float32)
    r1 = jnp.dot(ones_row, lt, preferred_element_type=jnp.float32)
    rle = jnp.dot(ones_row, le, preferred_element_type=jnp.float32)
    inv_e = 1.0 / (rle - r1)

    qmeans = []
    for kq in kqs:
        frac = jnp.clip((kq - r1) * inv_e, 0.0, 1.0)
        qmeans.append(
            jnp.sum(dmin_h * frac, axis=1, keepdims=True) * (1.0 / kq))

    mean_dh = jnp.sum(dmin_h, axis=1, keepdims=True) * (1.0 / nh)
    min_dh = jnp.min(dmin_h, axis=1, keepdims=True)
    mean_wh = jnp.sum(w_h, axis=1, keepdims=True) * (1.0 / nh)
    mean_do = jnp.sum(dmin_o, axis=1, keepdims=True) * (1.0 / no)

    return jnp.concatenate(
        [mean_dh, min_dh, qmeans[0], qmeans[1], qmeans[2],
         mean_wh, dir_mean, mean_do], axis=1)


def kernel(human_bt_n3, object_bt_m3, s_h_bt_n, s_o_bt_m, W1, b1, W2, b2):
    B, T, Nh, _ = human_bt_n3.shape
    No = object_bt_m3.shape[2]
    BT = B * T
    h = human_bt_n3.reshape(BT, Nh, 3)
    o = object_bt_m3.reshape(BT, No, 3)
    hoTm2 = jnp.concatenate([h, o], axis=1).transpose(0, 2, 1) * -2.0
    b2c = jnp.sum(o * o, axis=2)[:, :, None]
    shr = s_h_bt_n.reshape(BT, 1, Nh)
    kqs = tuple(int(max(1, round(q * Nh))) for q in (0.2, 0.5, 0.8))

    NS = 8
    feats = pl.pallas_call(
        functools.partial(_feats_body, nh=Nh, no=No, kqs=kqs, ns=NS),
        grid=(BT // NS,),
        in_specs=[
            pl.BlockSpec((NS, No, 3), lambda i: (i, 0, 0)),
            pl.BlockSpec((NS, 3, Nh + No), lambda i: (i, 0, 0)),
            pl.BlockSpec((NS, No, 1), lambda i: (i, 0, 0)),
            pl.BlockSpec((NS, 1, Nh), lambda i: (i, 0, 0)),
        ],
        out_specs=pl.BlockSpec((NS, 1, 10), lambda i: (i, 0, 0)),
        out_shape=jax.ShapeDtypeStruct((BT, 1, 10), jnp.float32),
    )(o, hoTm2, b2c, shr)
    feats = feats.reshape(BT, 10)

    H = W1.shape[1]
    F = W2.shape[1]
    out = pl.pallas_call(
        _mlp_body,
        in_specs=[pl.BlockSpec(feats.shape, lambda: (0, 0)),
                  pl.BlockSpec(W1.shape, lambda: (0, 0)),
                  pl.BlockSpec((1, H), lambda: (0, 0)),
                  pl.BlockSpec(W2.shape, lambda: (0, 0)),
                  pl.BlockSpec((1, F), lambda: (0, 0))],
        out_specs=pl.BlockSpec((BT, F), lambda: (0, 0)),
        out_shape=jax.ShapeDtypeStruct((BT, F), jnp.float32),
    )(feats, W1, b1.reshape(1, H), W2, b2.reshape(1, F))
    return out.reshape(B, T, F)


# speedup vs baseline: 1.3877x; 1.0011x over previous
import functools

import jax
import jax.numpy as jnp
from jax.experimental import pallas as pl
from jax.experimental.pallas import tpu as pltpu

_TAU = 0.05


def _feats_body(o_ref, hoT_ref, b2c_ref, sh_ref,
                f_ref, *, nh, no, kqs, ns):
    for s in range(ns):
        f_ref[s] = _one_sample(o_ref[s], hoT_ref[s], b2c_ref[s],
                               sh_ref[s], nh, no, kqs)


def _mlp_body(f_ref, w1_ref, b1_ref, w2_ref, b2_ref, out_ref):
    hid = jnp.maximum(
        jnp.dot(f_ref[...], w1_ref[...], preferred_element_type=jnp.float32)
        + b1_ref[...], 0.0)
    out_ref[...] = (
        jnp.dot(hid, w2_ref[...], preferred_element_type=jnp.float32)
        + b2_ref[...])


def _one_sample(o, hoTm2, b2c, shr, nh, no, kqs):
    hTm2 = hoTm2[:, :nh]
    oTm2 = hoTm2[:, nh:]
    a2r = jnp.sum(hTm2 * hTm2, axis=0, keepdims=True) * 0.25
    gT2 = jnp.dot(o, hTm2, preferred_element_type=jnp.float32)
    sqT = (b2c + a2r) + gT2

    sqmin_h = jnp.min(sqT, axis=0, keepdims=True)
    sqmin_o = jnp.min(sqT, axis=1, keepdims=True)
    dmin_h = jnp.sqrt(jnp.maximum(sqmin_h, 1e-12))
    dmin_o = jnp.sqrt(jnp.maximum(jnp.transpose(sqmin_o), 1e-12))

    eqf = (sqT == sqmin_h).astype(jnp.float32)
    o_nn = jnp.dot(oTm2, eqf,
                   preferred_element_type=jnp.float32) * -0.5
    cnt = jnp.dot(jnp.ones((1, no), jnp.float32), eqf,
                  preferred_element_type=jnp.float32)

    vecT = o_nn / cnt - hTm2 * -0.5
    nrm = jnp.sqrt(jnp.maximum(
        jnp.sum(vecT * vecT, axis=0, keepdims=True), 1e-6))
    dir_sum = jnp.sum(vecT / nrm, axis=1, keepdims=True)
    dir_mean = jnp.transpose(dir_sum) * (1.0 / nh)

    w_h = jnp.exp(dmin_h * (-1.0 / _TAU)) * shr

    dm_col = jnp.transpose(dmin_h)
    ones_row = jnp.ones((1, nh), jnp.float32)
    lt = (dm_col < dmin_h).astype(jnp.float32)
    le = (dm_col <= dmin_h).astype(jnp.float32)
    r1 = jnp.dot(ones_row, lt, preferred_element_type=jnp.float32)
    rle = jnp.dot(ones_row, le, preferred_element_type=jnp.float32)
    inv_e = 1.0 / (rle - r1)

    qmeans = []
    for kq in kqs:
        frac = jnp.clip((kq - r1) * inv_e, 0.0, 1.0)
        qmeans.append(
            jnp.sum(dmin_h * frac, axis=1, keepdims=True) * (1.0 / kq))

    mean_dh = jnp.sum(dmin_h, axis=1, keepdims=True) * (1.0 / nh)
    min_dh = jnp.min(dmin_h, axis=1, keepdims=True)
    mean_wh = jnp.sum(w_h, axis=1, keepdims=True) * (1.0 / nh)
    mean_do = jnp.sum(dmin_o, axis=1, keepdims=True) * (1.0 / no)

    return jnp.concatenate(
        [mean_dh, min_dh, qmeans[0], qmeans[1], qmeans[2],
         mean_wh, dir_mean, mean_do], axis=1)


def kernel(human_bt_n3, object_bt_m3, s_h_bt_n, s_o_bt_m, W1, b1, W2, b2):
    B, T, Nh, _ = human_bt_n3.shape
    No = object_bt_m3.shape[2]
    BT = B * T
    h = human_bt_n3.reshape(BT, Nh, 3)
    o = object_bt_m3.reshape(BT, No, 3)
    hoTm2 = jnp.concatenate([h, o], axis=1).transpose(0, 2, 1) * -2.0
    b2c = jnp.sum(o * o, axis=2)[:, :, None]
    shr = s_h_bt_n.reshape(BT, 1, Nh)
    kqs = tuple(int(max(1, round(q * Nh))) for q in (0.2, 0.5, 0.8))

    NS = 8
    feats = pl.pallas_call(
        functools.partial(_feats_body, nh=Nh, no=No, kqs=kqs, ns=NS),
        grid=(BT // NS,),
        in_specs=[
            pl.BlockSpec((NS, No, 3), lambda i: (i, 0, 0)),
            pl.BlockSpec((NS, 3, Nh + No), lambda i: (i, 0, 0)),
            pl.BlockSpec((NS, No, 1), lambda i: (i, 0, 0)),
            pl.BlockSpec((NS, 1, Nh), lambda i: (i, 0, 0)),
        ],
        out_specs=pl.BlockSpec((NS, 1, 10), lambda i: (i, 0, 0)),
        out_shape=jax.ShapeDtypeStruct((BT, 1, 10), jnp.float32),
        compiler_params=pltpu.CompilerParams(
            dimension_semantics=("parallel",)),
    )(o, hoTm2, b2c, shr)
    feats = feats.reshape(BT, 10)

    H = W1.shape[1]
    F = W2.shape[1]
    out = pl.pallas_call(
        _mlp_body,
        in_specs=[pl.BlockSpec(feats.shape, lambda: (0, 0)),
                  pl.BlockSpec(W1.shape, lambda: (0, 0)),
                  pl.BlockSpec((1, H), lambda: (0, 0)),
                  pl.BlockSpec(W2.shape, lambda: (0, 0)),
                  pl.BlockSpec((1, F), lambda: (0, 0))],
        out_specs=pl.BlockSpec((BT, F), lambda: (0, 0)),
        out_shape=jax.ShapeDtypeStruct((BT, F), jnp.float32),
    )(feats, W1, b1.reshape(1, H), W2, b2.reshape(1, F))
    return out.reshape(B, T, F)


# speedup vs baseline: 1.5048x; 1.0844x over previous
import functools

import jax
import jax.numpy as jnp
from jax.experimental import pallas as pl
from jax.experimental.pallas import tpu as pltpu

_TAU = 0.05


def _feats_body(o4_ref, hoT_ref, sh_ref,
                f_ref, *, nh, no, kqs, ns):
    for s in range(ns):
        f_ref[s] = _one_sample(o4_ref[s], hoT_ref[s],
                               sh_ref[s], nh, no, kqs)


def _mlp_body(f_ref, w1_ref, b1_ref, w2_ref, b2_ref, out_ref):
    hid = jnp.maximum(
        jnp.dot(f_ref[...], w1_ref[...], preferred_element_type=jnp.float32)
        + b1_ref[...], 0.0)
    out_ref[...] = (
        jnp.dot(hid, w2_ref[...], preferred_element_type=jnp.float32)
        + b2_ref[...])


def _one_sample(o4, hoTm2, shr, nh, no, kqs):
    o = o4[:, :3]
    b2c = o4[:, 3:4]
    hTm2 = hoTm2[:, :nh]
    oTm2 = hoTm2[:, nh:]
    a2r = jnp.sum(hTm2 * hTm2, axis=0, keepdims=True) * 0.25
    gT2 = jnp.dot(o, hTm2, preferred_element_type=jnp.float32)
    sqT = (b2c + a2r) + gT2

    sqmin_h = jnp.min(sqT, axis=0, keepdims=True)
    sqmin_o = jnp.min(sqT, axis=1, keepdims=True)
    dmin_h = jnp.sqrt(jnp.maximum(sqmin_h, 1e-12))
    dmin_o = jnp.sqrt(jnp.maximum(jnp.transpose(sqmin_o), 1e-12))

    eqf = (sqT == sqmin_h).astype(jnp.float32)
    o_nn = jnp.dot(oTm2, eqf,
                   preferred_element_type=jnp.float32) * -0.5
    cnt = jnp.dot(jnp.ones((1, no), jnp.float32), eqf,
                  preferred_element_type=jnp.float32)

    vecT = o_nn / cnt - hTm2 * -0.5
    nrm = jnp.sqrt(jnp.maximum(
        jnp.sum(vecT * vecT, axis=0, keepdims=True), 1e-6))
    dir_sum = jnp.sum(vecT / nrm, axis=1, keepdims=True)
    dir_mean = jnp.transpose(dir_sum) * (1.0 / nh)

    w_h = jnp.exp(dmin_h * (-1.0 / _TAU)) * shr

    dm_col = jnp.transpose(dmin_h)
    ones_row = jnp.ones((1, nh), jnp.float32)
    lt = (dm_col < dmin_h).astype(jnp.float32)
    le = (dm_col <= dmin_h).astype(jnp.float32)
    r1 = jnp.dot(ones_row, lt, preferred_element_type=jnp.float32)
    rle = jnp.dot(ones_row, le, preferred_element_type=jnp.float32)
    inv_e = 1.0 / (rle - r1)

    qmeans = []
    for kq in kqs:
        frac = jnp.clip((kq - r1) * inv_e, 0.0, 1.0)
        qmeans.append(
            jnp.sum(dmin_h * frac, axis=1, keepdims=True) * (1.0 / kq))

    mean_dh = jnp.sum(dmin_h, axis=1, keepdims=True) * (1.0 / nh)
    min_dh = jnp.min(dmin_h, axis=1, keepdims=True)
    mean_wh = jnp.sum(w_h, axis=1, keepdims=True) * (1.0 / nh)
    mean_do = jnp.sum(dmin_o, axis=1, keepdims=True) * (1.0 / no)

    return jnp.concatenate(
        [mean_dh, min_dh, qmeans[0], qmeans[1], qmeans[2],
         mean_wh, dir_mean, mean_do], axis=1)


def kernel(human_bt_n3, object_bt_m3, s_h_bt_n, s_o_bt_m, W1, b1, W2, b2):
    B, T, Nh, _ = human_bt_n3.shape
    No = object_bt_m3.shape[2]
    BT = B * T
    h = human_bt_n3.reshape(BT, Nh, 3)
    o = object_bt_m3.reshape(BT, No, 3)
    hoTm2 = jnp.concatenate([h, o], axis=1).transpose(0, 2, 1) * -2.0
    o4 = jnp.concatenate([o, jnp.sum(o * o, axis=2, keepdims=True)], axis=2)
    shr = s_h_bt_n.reshape(BT, 1, Nh)
    kqs = tuple(int(max(1, round(q * Nh))) for q in (0.2, 0.5, 0.8))

    NS = 8
    feats = pl.pallas_call(
        functools.partial(_feats_body, nh=Nh, no=No, kqs=kqs, ns=NS),
        grid=(BT // NS,),
        in_specs=[
            pl.BlockSpec((NS, No, 4), lambda i: (i, 0, 0)),
            pl.BlockSpec((NS, 3, Nh + No), lambda i: (i, 0, 0)),
            pl.BlockSpec((NS, 1, Nh), lambda i: (i, 0, 0)),
        ],
        out_specs=pl.BlockSpec((NS, 1, 10), lambda i: (i, 0, 0)),
        out_shape=jax.ShapeDtypeStruct((BT, 1, 10), jnp.float32),
        compiler_params=pltpu.CompilerParams(
            dimension_semantics=("parallel",)),
    )(o4, hoTm2, shr)
    feats = feats.reshape(BT, 10)

    H = W1.shape[1]
    F = W2.shape[1]
    out = pl.pallas_call(
        _mlp_body,
        in_specs=[pl.BlockSpec(feats.shape, lambda: (0, 0)),
                  pl.BlockSpec(W1.shape, lambda: (0, 0)),
                  pl.BlockSpec((1, H), lambda: (0, 0)),
                  pl.BlockSpec(W2.shape, lambda: (0, 0)),
                  pl.BlockSpec((1, F), lambda: (0, 0))],
        out_specs=pl.BlockSpec((BT, F), lambda: (0, 0)),
        out_shape=jax.ShapeDtypeStruct((BT, F), jnp.float32),
    )(feats, W1, b1.reshape(1, H), W2, b2.reshape(1, F))
    return out.reshape(B, T, F)
